# baseline (speedup 1.0000x reference)
import functools

import jax
import jax.numpy as jnp
from jax import lax
from jax.experimental import pallas as pl
from jax.experimental.pallas import tpu as pltpu
from jax.experimental.pallas import tpu_sc as plsc

N = 10000
G = 64
NEG_SLOPE = 0.2
E_REAL = 170000
NS = 16
NC = 2
KB = 128
NBLK = 84
E_PAD = NS * NBLK * KB
STRIPE = 640
TAIL = N - (NS - 1) * STRIPE
D = 512
CW = 128
NCH = D // CW
NBUF = 4
NPASS = NCH // NC


def _proj_body(x_ref, w_ref, a2_ref, h_ref, al_ref):
    j = pl.program_id(1)
    h = jnp.dot(x_ref[...], w_ref[...], preferred_element_type=jnp.float32)
    h_ref[0] = h.astype(jnp.bfloat16)

    @pl.when(j == 0)
    def _():
        al_ref[...] = jnp.zeros_like(al_ref)

    al_ref[...] += jnp.dot(h, a2_ref[...], preferred_element_type=jnp.float32)


def _project(x, W, a_src, a_dst):
    n, k = x.shape
    rb = 1000
    a2 = jnp.concatenate(
        [jnp.tile(a_src[:, None], (1, 64)), jnp.tile(a_dst[:, None], (1, 64))],
        axis=1)
    h4, al = pl.pallas_call(
        _proj_body,
        grid=(n // rb, NCH),
        in_specs=[
            pl.BlockSpec((rb, k), lambda i, j: (i, 0)),
            pl.BlockSpec((k, 128), lambda i, j: (0, j)),
            pl.BlockSpec((128, 128), lambda i, j: (j, 0)),
        ],
        out_specs=[
            pl.BlockSpec((1, rb, CW), lambda i, j: (j, i, 0)),
            pl.BlockSpec((rb, 128), lambda i, j: (i, 0)),
        ],
        out_shape=[
            jax.ShapeDtypeStruct((NCH, n, CW), jnp.bfloat16),
            jax.ShapeDtypeStruct((n, 128), jnp.float32),
        ],
    )(x, W, a2)
    return h4, al[:, 0], al[:, 64]


def _edge_body(src_hbm, dst_hbm, asrc_hbm, adst_hbm, h4_hbm,
               spart_hbm, u4_hbm,
               srcv, dstv, asrc_v, adst_v, p_v, s_tile, gbuf, acc_sh,
               gsems, ssems):
    c = lax.axis_index("c")
    s = lax.axis_index("s")

    pltpu.sync_copy(src_hbm.at[s], srcv)
    pltpu.sync_copy(dst_hbm.at[s], dstv)
    pltpu.sync_copy(asrc_hbm, asrc_v)
    pltpu.sync_copy(adst_hbm, adst_v)

    zero16 = jnp.zeros((16,), jnp.float32)

    def _zs(i, _):
        s_tile[pl.ds(i * 16, 16)] = zero16
        return 0
    lax.fori_loop(0, N // 16, _zs, 0)

    lanes = lax.iota(jnp.int32, 16)

    def _pa(r, _):
        for g2 in range(KB // 32):
            pp = []
            for g in (2 * g2, 2 * g2 + 1):
                si = srcv[r, pl.ds(g * 16, 16)]
                di = dstv[r, pl.ds(g * 16, 16)]
                a = plsc.load_gather(asrc_v, [si])
                b = plsc.load_gather(adst_v, [di])
                e = a + b
                e = jnp.where(e >= 0, e, e * NEG_SLOPE)
                p = jnp.exp(e)
                gid = (s * NBLK + r) * KB + g * 16 + lanes
                p = jnp.where(gid < E_REAL, p, 0.0)
                pp.append(p)
                plsc.addupdate_scatter(s_tile, [di], p)
            p_v[r, pl.ds(g2 * 16, 16)] = plsc.bitcast(
                plsc.pack(pp[0], pp[1], format=plsc.PackFormat.INTERLEAVED),
                jnp.int32)
        return 0
    lax.fori_loop(0, NBLK, _pa, 0)

    @pl.when(c == 0)
    def _():
        for i in range(N // 1000):
            pltpu.sync_copy(s_tile.at[pl.ds(i * 1000, 1000)],
                            spart_hbm.at[i].at[s])

    for chunk in range(NPASS):
        cid = c * NPASS + chunk

        zero32 = jnp.zeros((32,), jnp.bfloat16)

        def _zg(r, _):
            for g in range(CW // 32):
                gbuf[0, r, pl.ds(g * 32, 32)] = zero32
            return 0
        lax.fori_loop(0, KB, _zg, 0)

        @pl.when(s < NS - 1)
        def _():
            for q in range(STRIPE // KB):
                pltpu.sync_copy(
                    gbuf.at[0],
                    acc_sh.at[pl.ds(pl.multiple_of(s * STRIPE + q * KB, 8), KB)])

        @pl.when(s == NS - 1)
        def _():
            for q in range(TAIL // KB):
                pltpu.sync_copy(
                    gbuf.at[0], acc_sh.at[pl.ds((NS - 1) * STRIPE + q * KB, KB)])
            pltpu.sync_copy(
                gbuf.at[0].at[pl.ds(0, TAIL % KB)],
                acc_sh.at[pl.ds((NS - 1) * STRIPE + (TAIL // KB) * KB, TAIL % KB)])
        plsc.subcore_barrier()

        for b in range(NBUF):
            pltpu.async_copy(h4_hbm.at[cid].at[srcv.at[b]], gbuf.at[b], gsems[b])

        def _pb(j4, _):
            for b in range(NBUF):
                blk = j4 * NBUF + b
                pltpu.make_async_copy(
                    h4_hbm.at[cid].at[srcv.at[blk]], gbuf.at[b], gsems[b]).wait()

                def _scale(q, _):
                    pv = p_v[blk, pl.ds(q * 16, 16)]
                    for ll in range(32):
                        wi = pv[ll % 16]
                        if ll < 16:
                            w2 = (wi & jnp.int32(65535)) | (wi << 16)
                        else:
                            w2 = (wi & jnp.int32(-65536)) | (
                                (wi >> 16) & jnp.int32(65535))
                        psb = plsc.bitcast(jnp.full((16,), w2, jnp.int32),
                                           jnp.bfloat16)
                        row = q * 32 + ll
                        for g in range(CW // 32):
                            gbuf[b, row, pl.ds(g * 32, 32)] = (
                                gbuf[b, row, pl.ds(g * 32, 32)] * psb)
                    return 0
                lax.fori_loop(0, KB // 32, _scale, 0, unroll=4)
                pltpu.async_copy(gbuf.at[b], acc_sh.at[dstv.at[blk]], ssems[b],
                                 add=True)

                pb = (b - 1) % NBUF
                pblk = blk - 1

                @pl.when((pblk >= 0) & (pblk + NBUF < NBLK))
                def _():
                    pltpu.make_async_copy(
                        gbuf.at[pb], acc_sh.at[dstv.at[pblk]], ssems[pb]).wait()
                    pltpu.async_copy(h4_hbm.at[cid].at[srcv.at[pblk + NBUF]],
                                     gbuf.at[pb], gsems[pb])
            return 0
        lax.fori_loop(0, NBLK // NBUF, _pb, 0)
        for b in range(NBUF):
            pltpu.make_async_copy(
                gbuf.at[b], acc_sh.at[dstv.at[NBLK - NBUF + b]], ssems[b]).wait()
        plsc.subcore_barrier()

        @pl.when(s < NS - 1)
        def _():
            off = pl.multiple_of(s * STRIPE, 8)
            pltpu.sync_copy(acc_sh.at[pl.ds(off, STRIPE)],
                            u4_hbm.at[cid].at[pl.ds(off, STRIPE)])

        @pl.when(s == NS - 1)
        def _():
            pltpu.sync_copy(acc_sh.at[pl.ds((NS - 1) * STRIPE, TAIL)],
                            u4_hbm.at[cid].at[pl.ds((NS - 1) * STRIPE, TAIL)])
        plsc.subcore_barrier()


def _edge_aggregate(src3, dst3, asrc, adst, h4):
    mesh = plsc.VectorSubcoreMesh(core_axis_name="c", subcore_axis_name="s",
                                  num_cores=NC, num_subcores=NS)
    f = pl.kernel(
        _edge_body,
        out_type=[
            jax.ShapeDtypeStruct((N // 1000, NS, 1000), jnp.float32),
            jax.ShapeDtypeStruct((NCH, N, CW), jnp.bfloat16),
        ],
        mesh=mesh,
        compiler_params=pltpu.CompilerParams(needs_layout_passes=False,
                                             use_tc_tiling_on_sc=False),
        scratch_types=[
            pltpu.VMEM((NBLK, KB), jnp.int32),
            pltpu.VMEM((NBLK, KB), jnp.int32),
            pltpu.VMEM((N,), jnp.float32),
            pltpu.VMEM((N,), jnp.float32),
            pltpu.VMEM((NBLK, KB // 2), jnp.int32),
            pltpu.VMEM((N,), jnp.float32),
            pltpu.VMEM((NBUF, KB, CW), jnp.bfloat16),
            pltpu.VMEM_SHARED((N, CW), jnp.bfloat16),
            [pltpu.SemaphoreType.DMA] * NBUF,
            [pltpu.SemaphoreType.DMA] * NBUF,
        ],
    )
    return f(src3, dst3, asrc, adst, h4)


def _epi_body(nrb, with_proj, u_ref, sp_ref, b_ref, batch_ref, w_ref, a2_ref,
              emb_ref, *rest):
    if with_proj:
        h_ref, al_ref, emb_acc, cnt_acc = rest
    else:
        emb_acc, cnt_acc = rest
    i = pl.program_id(0)

    s = jnp.sum(sp_ref[0], axis=0, keepdims=True) + 1e-16
    u = jnp.concatenate([u_ref[c] for c in range(NCH)],
                        axis=-1).astype(jnp.float32)
    g = jnp.maximum(u / s.T + b_ref[...], 0.0)

    onehot = (batch_ref[0] == lax.broadcasted_iota(jnp.int32, (G, 1), 0)
              ).astype(jnp.float32)

    @pl.when(i == 0)
    def _():
        emb_acc[...] = jnp.zeros_like(emb_acc)
        cnt_acc[...] = jnp.zeros_like(cnt_acc)

    emb_acc[...] += jnp.dot(onehot, g, preferred_element_type=jnp.float32)
    cnt_acc[...] += jnp.broadcast_to(
        jnp.sum(onehot, axis=1, keepdims=True), cnt_acc.shape)

    @pl.when(i == nrb - 1)
    def _():
        emb_ref[...] = emb_acc[...] / jnp.maximum(cnt_acc[:, 0:1], 1.0)

    if with_proj:
        h = jnp.dot(g, w_ref[...], preferred_element_type=jnp.float32)
        for cch in range(NCH):
            h_ref[cch] = h[:, cch * CW:(cch + 1) * CW].astype(jnp.bfloat16)
        al_ref[...] = jnp.dot(h, a2_ref[...], preferred_element_type=jnp.float32)


def _epilogue(u4, s_part, b, batch3, W=None, a_src=None, a_dst=None):
    rb = 1000
    nrb = N // rb
    with_proj = W is not None
    if with_proj:
        a2 = jnp.concatenate(
            [jnp.tile(a_src[:, None], (1, 64)), jnp.tile(a_dst[:, None], (1, 64))],
            axis=1)
        w_in, a2_in = W, a2
    else:
        w_in = jnp.zeros((8, 128), jnp.float32)
        a2_in = jnp.zeros((8, 128), jnp.float32)
    kd, wd = w_in.shape

    out_specs = [pl.BlockSpec((G, D), lambda i: (0, 0))]
    out_shape = [jax.ShapeDtypeStruct((G, D), jnp.float32)]
    if with_proj:
        out_specs += [
            pl.BlockSpec((NCH, rb, CW), lambda i: (0, i, 0)),
            pl.BlockSpec((rb, 128), lambda i: (i, 0)),
        ]
        out_shape += [
            jax.ShapeDtypeStruct((NCH, N, CW), jnp.bfloat16),
            jax.ShapeDtypeStruct((N, 128), jnp.float32),
        ]
    res = pl.pallas_call(
        functools.partial(_epi_body, nrb, with_proj),
        grid=(nrb,),
        in_specs=[
            pl.BlockSpec((NCH, rb, CW), lambda i: (0, i, 0)),
            pl.BlockSpec((1, NS, rb), lambda i: (i, 0, 0)),
            pl.BlockSpec((1, D), lambda i: (0, 0)),
            pl.BlockSpec((1, 1, rb), lambda i: (i, 0, 0)),
            pl.BlockSpec((kd, wd), lambda i: (0, 0)),
            pl.BlockSpec((kd, 128), lambda i: (0, 0)),
        ],
        out_specs=out_specs,
        out_shape=out_shape,
        scratch_shapes=[
            pltpu.VMEM((G, D), jnp.float32),
            pltpu.VMEM((G, 128), jnp.float32),
        ],
    )(u4, s_part, b.reshape(1, D), batch3, w_in, a2_in)
    if with_proj:
        emb, h4, al = res
        return emb, h4, al[:, 0], al[:, 64]
    return res[0]


def kernel(x, edge_index, batch, W1, a_src1, a_dst1, b1, W2, a_src2, a_dst2, b2):
    loop = jnp.arange(N, dtype=edge_index.dtype)
    ei = jnp.concatenate([edge_index, jnp.stack([loop, loop])], axis=1)
    pad = jnp.zeros((2, E_PAD - E_REAL), dtype=ei.dtype)
    ei = jnp.concatenate([ei, pad], axis=1)
    src3 = ei[0].reshape(NS, NBLK, KB)
    dst3 = ei[1].reshape(NS, NBLK, KB)
    batch3 = batch.reshape(10, 1, 1000)

    h4, asrc, adst = _project(x, W1, a_src1, a_dst1)
    s_part, u4 = _edge_aggregate(src3, dst3, asrc, adst, h4)
    emb1, h4b, asrc2v, adst2v = _epilogue(u4, s_part, b1, batch3,
                                          W2, a_src2, a_dst2)
    s_part2, u4b = _edge_aggregate(src3, dst3, asrc2v, adst2v, h4b)
    emb2 = _epilogue(u4b, s_part2, b2, batch3)
    return (emb1, emb2)


# speedup vs baseline: 1.0376x; 1.0376x over previous
import functools

import jax
import jax.numpy as jnp
from jax import lax
from jax.experimental import pallas as pl
from jax.experimental.pallas import tpu as pltpu
from jax.experimental.pallas import tpu_sc as plsc

N = 10000
G = 64
NEG_SLOPE = 0.2
E_REAL = 170000
NS = 16
NC = 2
KB = 128
NBLK = 84
E_PAD = NS * NBLK * KB
STRIPE = 640
TAIL = N - (NS - 1) * STRIPE
D = 512
CW = 128
NCH = D // CW
NBUF = 4
NPASS = NCH // NC


def _proj_body(x_ref, w_ref, a2_ref, h_ref, al_ref):
    j = pl.program_id(1)
    h = jnp.dot(x_ref[...], w_ref[...], preferred_element_type=jnp.float32)
    h_ref[0] = h.astype(jnp.bfloat16)

    @pl.when(j == 0)
    def _():
        al_ref[...] = jnp.zeros_like(al_ref)

    al_ref[...] += jnp.dot(h, a2_ref[...], preferred_element_type=jnp.float32)


def _project(x, W, a_src, a_dst):
    n, k = x.shape
    rb = 1000
    a2 = jnp.concatenate(
        [jnp.tile(a_src[:, None], (1, 64)), jnp.tile(a_dst[:, None], (1, 64))],
        axis=1)
    h4, al = pl.pallas_call(
        _proj_body,
        grid=(n // rb, NCH),
        in_specs=[
            pl.BlockSpec((rb, k), lambda i, j: (i, 0)),
            pl.BlockSpec((k, 128), lambda i, j: (0, j)),
            pl.BlockSpec((128, 128), lambda i, j: (j, 0)),
        ],
        out_specs=[
            pl.BlockSpec((1, rb, CW), lambda i, j: (j, i, 0)),
            pl.BlockSpec((rb, 128), lambda i, j: (i, 0)),
        ],
        out_shape=[
            jax.ShapeDtypeStruct((NCH, n, CW), jnp.bfloat16),
            jax.ShapeDtypeStruct((n, 128), jnp.float32),
        ],
    )(x, W, a2)
    return h4, al[:, 0], al[:, 64]


def _edge_body(src_hbm, dst_hbm, asrc_hbm, adst_hbm, h4_hbm,
               spart_hbm, u4_hbm,
               srcv, dstv, asrc_v, adst_v, p_v, s_tile, gbuf, acc_sh,
               gsems, ssems):
    c = lax.axis_index("c")
    s = lax.axis_index("s")

    pltpu.sync_copy(src_hbm.at[s], srcv)
    pltpu.sync_copy(dst_hbm.at[s], dstv)
    pltpu.sync_copy(asrc_hbm, asrc_v)
    pltpu.sync_copy(adst_hbm, adst_v)

    zero16 = jnp.zeros((16,), jnp.float32)

    def _zs(i, _):
        s_tile[pl.ds(i * 16, 16)] = zero16
        return 0
    lax.fori_loop(0, N // 16, _zs, 0)

    lanes = lax.iota(jnp.int32, 16)

    def _pa(r, _):
        for g2 in range(KB // 32):
            pp = []
            for g in (2 * g2, 2 * g2 + 1):
                si = srcv[r, pl.ds(g * 16, 16)]
                di = dstv[r, pl.ds(g * 16, 16)]
                a = plsc.load_gather(asrc_v, [si])
                b = plsc.load_gather(adst_v, [di])
                e = a + b
                e = jnp.where(e >= 0, e, e * NEG_SLOPE)
                p = jnp.exp(e)
                gid = (s * NBLK + r) * KB + g * 16 + lanes
                p = jnp.where(gid < E_REAL, p, 0.0)
                pp.append(p)
                plsc.addupdate_scatter(s_tile, [di], p)
            p_v[r, pl.ds(g2 * 16, 16)] = plsc.bitcast(
                plsc.pack(pp[0], pp[1], format=plsc.PackFormat.INTERLEAVED),
                jnp.int32)
        return 0
    lax.fori_loop(0, NBLK, _pa, 0)

    @pl.when(c == 0)
    def _():
        for i in range(N // 1000):
            pltpu.sync_copy(s_tile.at[pl.ds(i * 1000, 1000)],
                            spart_hbm.at[i].at[s])

    for chunk in range(NPASS):
        cid = c * NPASS + chunk

        zero32 = jnp.zeros((32,), jnp.bfloat16)

        def _zg(r, _):
            for g in range(CW // 32):
                gbuf[0, r, pl.ds(g * 32, 32)] = zero32
            return 0
        lax.fori_loop(0, KB, _zg, 0)

        @pl.when(s < NS - 1)
        def _():
            for q in range(STRIPE // KB):
                pltpu.sync_copy(
                    gbuf.at[0],
                    acc_sh.at[pl.ds(pl.multiple_of(s * STRIPE + q * KB, 8), KB)])

        @pl.when(s == NS - 1)
        def _():
            for q in range(TAIL // KB):
                pltpu.sync_copy(
                    gbuf.at[0], acc_sh.at[pl.ds((NS - 1) * STRIPE + q * KB, KB)])
            pltpu.sync_copy(
                gbuf.at[0].at[pl.ds(0, TAIL % KB)],
                acc_sh.at[pl.ds((NS - 1) * STRIPE + (TAIL // KB) * KB, TAIL % KB)])
        plsc.subcore_barrier()

        for b in range(NBUF):
            pltpu.async_copy(h4_hbm.at[cid].at[srcv.at[b]], gbuf.at[b], gsems[b])

        def _pb(j4, _):
            for b in range(NBUF):
                blk = j4 * NBUF + b
                pltpu.make_async_copy(
                    h4_hbm.at[cid].at[srcv.at[blk]], gbuf.at[b], gsems[b]).wait()

                def _scale(q, _):
                    pv = p_v[blk, pl.ds(q * 16, 16)]
                    for ll in range(32):
                        wi = pv[ll % 16]
                        if ll < 16:
                            w2 = (wi & jnp.int32(65535)) | (wi << 16)
                        else:
                            w2 = (wi & jnp.int32(-65536)) | (
                                (wi >> 16) & jnp.int32(65535))
                        psb = plsc.bitcast(jnp.full((16,), w2, jnp.int32),
                                           jnp.bfloat16)
                        row = q * 32 + ll
                        for g in range(CW // 32):
                            gbuf[b, row, pl.ds(g * 32, 32)] = (
                                gbuf[b, row, pl.ds(g * 32, 32)] * psb)
                    return 0
                lax.fori_loop(0, KB // 32, _scale, 0)
                pltpu.async_copy(gbuf.at[b], acc_sh.at[dstv.at[blk]], ssems[b],
                                 add=True)

                pb = (b - 1) % NBUF
                pblk = blk - 1

                @pl.when((pblk >= 0) & (pblk + NBUF < NBLK))
                def _():
                    pltpu.make_async_copy(
                        gbuf.at[pb], acc_sh.at[dstv.at[pblk]], ssems[pb]).wait()
                    pltpu.async_copy(h4_hbm.at[cid].at[srcv.at[pblk + NBUF]],
                                     gbuf.at[pb], gsems[pb])
            return 0
        lax.fori_loop(0, NBLK // NBUF, _pb, 0)
        for b in range(NBUF):
            pltpu.make_async_copy(
                gbuf.at[b], acc_sh.at[dstv.at[NBLK - NBUF + b]], ssems[b]).wait()
        plsc.subcore_barrier()

        @pl.when(s < NS - 1)
        def _():
            off = pl.multiple_of(s * STRIPE, 8)
            pltpu.sync_copy(acc_sh.at[pl.ds(off, STRIPE)],
                            u4_hbm.at[cid].at[pl.ds(off, STRIPE)])

        @pl.when(s == NS - 1)
        def _():
            pltpu.sync_copy(acc_sh.at[pl.ds((NS - 1) * STRIPE, TAIL)],
                            u4_hbm.at[cid].at[pl.ds((NS - 1) * STRIPE, TAIL)])
        plsc.subcore_barrier()


def _edge_aggregate(src3, dst3, asrc, adst, h4):
    mesh = plsc.VectorSubcoreMesh(core_axis_name="c", subcore_axis_name="s",
                                  num_cores=NC, num_subcores=NS)
    f = pl.kernel(
        _edge_body,
        out_type=[
            jax.ShapeDtypeStruct((N // 1000, NS, 1000), jnp.float32),
            jax.ShapeDtypeStruct((NCH, N, CW), jnp.bfloat16),
        ],
        mesh=mesh,
        compiler_params=pltpu.CompilerParams(needs_layout_passes=False,
                                             use_tc_tiling_on_sc=False),
        scratch_types=[
            pltpu.VMEM((NBLK, KB), jnp.int32),
            pltpu.VMEM((NBLK, KB), jnp.int32),
            pltpu.VMEM((N,), jnp.float32),
            pltpu.VMEM((N,), jnp.float32),
            pltpu.VMEM((NBLK, KB // 2), jnp.int32),
            pltpu.VMEM((N,), jnp.float32),
            pltpu.VMEM((NBUF, KB, CW), jnp.bfloat16),
            pltpu.VMEM_SHARED((N, CW), jnp.bfloat16),
            [pltpu.SemaphoreType.DMA] * NBUF,
            [pltpu.SemaphoreType.DMA] * NBUF,
        ],
    )
    return f(src3, dst3, asrc, adst, h4)


def _epi_body(nrb, with_proj, u_ref, sp_ref, b_ref, batch_ref, w_ref, a2_ref,
              emb_ref, *rest):
    if with_proj:
        h_ref, al_ref, emb_acc, cnt_acc = rest
    else:
        emb_acc, cnt_acc = rest
    i = pl.program_id(0)

    s = jnp.sum(sp_ref[0], axis=0, keepdims=True) + 1e-16
    u = jnp.concatenate([u_ref[c] for c in range(NCH)],
                        axis=-1).astype(jnp.float32)
    g = jnp.maximum(u / s.T + b_ref[...], 0.0)

    onehot = (batch_ref[0] == lax.broadcasted_iota(jnp.int32, (G, 1), 0)
              ).astype(jnp.float32)

    @pl.when(i == 0)
    def _():
        emb_acc[...] = jnp.zeros_like(emb_acc)
        cnt_acc[...] = jnp.zeros_like(cnt_acc)

    emb_acc[...] += jnp.dot(onehot, g, preferred_element_type=jnp.float32)
    cnt_acc[...] += jnp.broadcast_to(
        jnp.sum(onehot, axis=1, keepdims=True), cnt_acc.shape)

    @pl.when(i == nrb - 1)
    def _():
        emb_ref[...] = emb_acc[...] / jnp.maximum(cnt_acc[:, 0:1], 1.0)

    if with_proj:
        h = jnp.dot(g, w_ref[...], preferred_element_type=jnp.float32)
        for cch in range(NCH):
            h_ref[cch] = h[:, cch * CW:(cch + 1) * CW].astype(jnp.bfloat16)
        al_ref[...] = jnp.dot(h, a2_ref[...], preferred_element_type=jnp.float32)


def _epilogue(u4, s_part, b, batch3, W=None, a_src=None, a_dst=None):
    rb = 1000
    nrb = N // rb
    with_proj = W is not None
    if with_proj:
        a2 = jnp.concatenate(
            [jnp.tile(a_src[:, None], (1, 64)), jnp.tile(a_dst[:, None], (1, 64))],
            axis=1)
        w_in, a2_in = W, a2
    else:
        w_in = jnp.zeros((8, 128), jnp.float32)
        a2_in = jnp.zeros((8, 128), jnp.float32)
    kd, wd = w_in.shape

    out_specs = [pl.BlockSpec((G, D), lambda i: (0, 0))]
    out_shape = [jax.ShapeDtypeStruct((G, D), jnp.float32)]
    if with_proj:
        out_specs += [
            pl.BlockSpec((NCH, rb, CW), lambda i: (0, i, 0)),
            pl.BlockSpec((rb, 128), lambda i: (i, 0)),
        ]
        out_shape += [
            jax.ShapeDtypeStruct((NCH, N, CW), jnp.bfloat16),
            jax.ShapeDtypeStruct((N, 128), jnp.float32),
        ]
    res = pl.pallas_call(
        functools.partial(_epi_body, nrb, with_proj),
        grid=(nrb,),
        in_specs=[
            pl.BlockSpec((NCH, rb, CW), lambda i: (0, i, 0)),
            pl.BlockSpec((1, NS, rb), lambda i: (i, 0, 0)),
            pl.BlockSpec((1, D), lambda i: (0, 0)),
            pl.BlockSpec((1, 1, rb), lambda i: (i, 0, 0)),
            pl.BlockSpec((kd, wd), lambda i: (0, 0)),
            pl.BlockSpec((kd, 128), lambda i: (0, 0)),
        ],
        out_specs=out_specs,
        out_shape=out_shape,
        scratch_shapes=[
            pltpu.VMEM((G, D), jnp.float32),
            pltpu.VMEM((G, 128), jnp.float32),
        ],
    )(u4, s_part, b.reshape(1, D), batch3, w_in, a2_in)
    if with_proj:
        emb, h4, al = res
        return emb, h4, al[:, 0], al[:, 64]
    return res[0]


def kernel(x, edge_index, batch, W1, a_src1, a_dst1, b1, W2, a_src2, a_dst2, b2):
    loop = jnp.arange(N, dtype=edge_index.dtype)
    ei = jnp.concatenate([edge_index, jnp.stack([loop, loop])], axis=1)
    pad = jnp.zeros((2, E_PAD - E_REAL), dtype=ei.dtype)
    ei = jnp.concatenate([ei, pad], axis=1)
    src3 = ei[0].reshape(NS, NBLK, KB)
    dst3 = ei[1].reshape(NS, NBLK, KB)
    batch3 = batch.reshape(10, 1, 1000)

    h4, asrc, adst = _project(x, W1, a_src1, a_dst1)
    s_part, u4 = _edge_aggregate(src3, dst3, asrc, adst, h4)
    emb1, h4b, asrc2v, adst2v = _epilogue(u4, s_part, b1, batch3,
                                          W2, a_src2, a_dst2)
    s_part2, u4b = _edge_aggregate(src3, dst3, asrc2v, adst2v, h4b)
    emb2 = _epilogue(u4b, s_part2, b2, batch3)
    return (emb1, emb2)
